# baseline (speedup 1.0000x reference)
import functools

import jax
import jax.numpy as jnp
from jax import lax
from jax.experimental import pallas as pl
from jax.experimental.pallas import tpu as pltpu
from jax.experimental.pallas import tpu_sc as plsc

_MARGIN = 1.0
_EPS = 1e-6
_L = 16
_D = 64
_NW = 32


def _sqrt16(x):
    x = jnp.maximum(x, 0.0)
    xc = jnp.maximum(x, jnp.float32(1e-20))
    i = plsc.bitcast(xc, jnp.int32)
    i = jnp.int32(0x5F3759DF) - lax.shift_right_logical(i, 1)
    r = plsc.bitcast(i, jnp.float32)
    for _ in range(3):
        r = r * (jnp.float32(1.5) - jnp.float32(0.5) * xc * r * r)
    return x * r


def _ranks16(pv):
    one = jnp.full((_L,), 1.0, jnp.float32)
    zero = jnp.zeros((_L,), jnp.float32)
    rk = [zero] * 20
    for c in range(20):
        for cc in range(c + 1, 20):
            ge = pv[c] >= pv[cc]
            rk[cc] = rk[cc] + jnp.where(ge, one, zero)
            rk[c] = rk[c] + jnp.where(ge, zero, one)
    return rk


def _bfull(v):
    return jnp.full((_L,), v, jnp.int32)


_DN = lax.GatherDimensionNumbers(
    offset_dims=(), collapsed_slice_dims=(0,), start_index_map=(0,))


def _lane_bcast(v, lane):
    return lax.gather(v, _bfull(lane)[:, None], _DN, (1,),
                      mode=lax.GatherScatterMode.PROMISE_IN_BOUNDS)


def _sc_partials(rows_per_w, stu_f, t1_f, t2_f, cw_f, w32):
    n_chunk = rows_per_w // _L
    mesh = plsc.VectorSubcoreMesh(core_axis_name="c", subcore_axis_name="s")

    @functools.partial(
        pl.kernel,
        mesh=mesh,
        out_type=jax.ShapeDtypeStruct((_NW, _L), jnp.float32),
        compiler_params=pltpu.CompilerParams(needs_layout_passes=False),
        scratch_types=[
            pltpu.VMEM((rows_per_w * _D,), jnp.float32),
            pltpu.VMEM((rows_per_w * 20,), jnp.float32),
            pltpu.VMEM((rows_per_w * 20,), jnp.float32),
            pltpu.VMEM((48 * _D,), jnp.float32),
            pltpu.VMEM((_D, 48), jnp.float32),
            pltpu.VMEM((32,), jnp.float32),
            pltpu.VMEM((32,), jnp.float32),
            pltpu.VMEM((48,), jnp.float32),
            pltpu.VMEM((_L * 48,), jnp.float32),
            pltpu.VMEM((20, _L), jnp.float32),
            pltpu.VMEM((20, _L), jnp.float32),
            pltpu.VMEM((20, _L), jnp.float32),
            pltpu.VMEM((20, _L), jnp.float32),
            pltpu.VMEM((_L,), jnp.float32),
        ],
    )
    def k(stu_h, t1_h, t2_h, cw_h, w_h, out_h,
          stu_v, t1_v, t2_v, cw_v, cwT_v, w_v, wt_v, cne_v,
          dv, p1T, p2T, w1b, w2b, acc_v):
        wid = lax.axis_index("s") * 2 + lax.axis_index("c")
        base = wid * rows_per_w
        pltpu.sync_copy(stu_h.at[pl.ds(base * _D, rows_per_w * _D)], stu_v)
        pltpu.sync_copy(t1_h.at[pl.ds(base * 20, rows_per_w * 20)], t1_v)
        pltpu.sync_copy(t2_h.at[pl.ds(base * 20, rows_per_w * 20)], t2_v)
        pltpu.sync_copy(cw_h, cw_v)
        pltpu.sync_copy(w_h, w_v)

        lanes = lax.iota(jnp.int32, _L)
        zero = jnp.zeros((_L,), jnp.float32)
        one = jnp.full((_L,), 1.0, jnp.float32)
        eps = jnp.float32(_EPS)

        for j in range(2):
            v = w_v[pl.ds(j * _L, _L)]
            s = one / (one + jnp.exp(-v))
            if j == 0:
                s = jnp.where(lanes == 0, zero, s)
            wt_v[pl.ds(j * _L, _L)] = s

        for j in range(3):
            cflat = (jnp.int32(j * _L) + lanes) * _D

            def cw_body(d, carry):
                sq, sm = carry
                col = plsc.load_gather(cw_v, [cflat + d])
                cwT_v[d, pl.ds(j * _L, _L)] = col
                return sq + col * col, sm + col

            sq, sm = lax.fori_loop(0, _D, cw_body, (zero, zero))
            cne_v[pl.ds(j * _L, _L)] = sq - (jnp.float32(2.0) * eps) * sm

        def chunk_body(ch, loss_acc):
            rbase = ch * _L
            ridx = rbase + lanes

            an2s = []
            for b in range(16):
                sq = zero
                sm = zero
                for kk in range(4):
                    v = stu_v[pl.ds((rbase + b) * _D + kk * _L, _L)]
                    sq = sq + v * v
                    sm = sm + v
                an2s.append(jnp.sum(sq) + (jnp.float32(2.0) * eps) * jnp.sum(sm)
                            + jnp.float32(_D) * eps * eps)

            for sb in range(2):
                def mac(db, accs):
                    out = list(accs)
                    rvs = [stu_v[pl.ds((rbase + sb * 8 + b) * _D + db * _L, _L)]
                           for b in range(8)]
                    for t in range(_L):
                        cws = [cwT_v[db * _L + t, pl.ds(j * _L, _L)]
                               for j in range(3)]
                        for b in range(8):
                            bv = _lane_bcast(rvs[b], t)
                            for j in range(3):
                                out[b * 3 + j] = out[b * 3 + j] + bv * cws[j]
                    return tuple(out)

                dots = lax.fori_loop(0, 4, mac, (zero,) * 24)
                for b in range(8):
                    an2 = jnp.full((_L,), an2s[sb * 8 + b], jnp.float32)
                    for j in range(3):
                        d2 = an2 + cne_v[pl.ds(j * _L, _L)] \
                            - jnp.float32(2.0) * dots[b * 3 + j]
                        dv[pl.ds((sb * 8 + b) * 48 + j * _L, _L)] = _sqrt16(d2)

            rflat20 = ridx * 20

            def t_p(c, _):
                p1T[c, :] = plsc.load_gather(t1_v, [rflat20 + c])
                p2T[c, :] = plsc.load_gather(t2_v, [rflat20 + c])
                return 0

            lax.fori_loop(0, 20, t_p, 0)

            lanes48 = lanes * 48

            r2 = _ranks16([p2T[c, :] for c in range(20)])
            dpos1 = zero
            for c in range(20):
                w2b[c, :] = plsc.load_gather(wt_v, [r2[c].astype(jnp.int32)])
                dcol = plsc.load_gather(dv, [lanes48 + (20 + c)])
                dpos1 = dpos1 + jnp.where(r2[c] == 0.0, dcol, zero)

            r1 = _ranks16([p1T[c, :] for c in range(20)])
            dpos2 = zero
            for c in range(20):
                w1b[c, :] = plsc.load_gather(wt_v, [r1[c].astype(jnp.int32)])
                dcol = plsc.load_gather(dv, [lanes48 + c])
                dpos2 = dpos2 + jnp.where(r1[c] == 0.0, dcol, zero)

            m = jnp.full((_L,), _MARGIN, jnp.float32)
            for c in range(20):
                d1 = plsc.load_gather(dv, [lanes48 + c])
                d2c = plsc.load_gather(dv, [lanes48 + (20 + c)])
                loss_acc = loss_acc + w1b[c, :] * jnp.maximum(
                    dpos1 - d1 + m, zero)
                loss_acc = loss_acc + w2b[c, :] * jnp.maximum(
                    dpos2 - d2c + m, zero)
            return loss_acc

        acc = lax.fori_loop(0, n_chunk, chunk_body, zero)
        acc_v[...] = acc
        pltpu.sync_copy(acc_v, out_h.at[wid])

    return k(stu_f, t1_f, t2_f, cw_f, w32)


def _final_sum(parts, inv_b):
    def body(x_ref, o_ref):
        o_ref[...] = jnp.sum(x_ref[...], keepdims=True).reshape(1, 1) \
            * jnp.float32(inv_b)

    return pl.pallas_call(
        body, out_shape=jax.ShapeDtypeStruct((1, 1), jnp.float32)
    )(parts)


def kernel(stu_emb, t1_prob, t2_prob, classifier_weight, weights):
    b = stu_emb.shape[0]
    cw = lax.stop_gradient(classifier_weight)
    cw48 = jnp.zeros((48, _D), jnp.float32).at[:40].set(cw)
    w32 = jnp.zeros((32,), jnp.float32).at[:20].set(weights)
    parts = _sc_partials(
        b // _NW, stu_emb.reshape(-1), t1_prob.reshape(-1),
        t2_prob.reshape(-1), cw48.reshape(-1), w32)
    return _final_sum(parts, 1.0 / b)[0, 0]


# speedup vs baseline: 1.4640x; 1.4640x over previous
import functools

import jax
import jax.numpy as jnp
from jax import lax
from jax.experimental import pallas as pl
from jax.experimental.pallas import tpu as pltpu
from jax.experimental.pallas import tpu_sc as plsc

_MARGIN = 1.0
_EPS = 1e-6
_L = 16
_D = 64
_NW = 32


def _sqrt16(x):
    x = jnp.maximum(x, 0.0)
    xc = jnp.maximum(x, jnp.float32(1e-20))
    i = plsc.bitcast(xc, jnp.int32)
    i = jnp.int32(0x5F3759DF) - lax.shift_right_logical(i, 1)
    r = plsc.bitcast(i, jnp.float32)
    for _ in range(3):
        r = r * (jnp.float32(1.5) - jnp.float32(0.5) * xc * r * r)
    return x * r


def _ranks16(pv):
    one = jnp.full((_L,), 1.0, jnp.float32)
    zero = jnp.zeros((_L,), jnp.float32)
    rk = [zero] * 20
    for c in range(20):
        for cc in range(c + 1, 20):
            ge = pv[c] >= pv[cc]
            rk[cc] = rk[cc] + jnp.where(ge, one, zero)
            rk[c] = rk[c] + jnp.where(ge, zero, one)
    return rk


def _bfull(v):
    return jnp.full((_L,), v, jnp.int32)


_DN = lax.GatherDimensionNumbers(
    offset_dims=(), collapsed_slice_dims=(0,), start_index_map=(0,))


def _lane_bcast(v, lane):
    return lax.gather(v, _bfull(lane)[:, None], _DN, (1,),
                      mode=lax.GatherScatterMode.PROMISE_IN_BOUNDS)


def _sc_partials(rows_per_w, stu_f, t1_f, t2_f, cw_f, w32):
    n_chunk = rows_per_w // _L
    mesh = plsc.VectorSubcoreMesh(core_axis_name="c", subcore_axis_name="s")

    @functools.partial(
        pl.kernel,
        mesh=mesh,
        out_type=jax.ShapeDtypeStruct((_NW, _L), jnp.float32),
        compiler_params=pltpu.CompilerParams(needs_layout_passes=False),
        scratch_types=[
            pltpu.VMEM((rows_per_w * _D,), jnp.float32),
            pltpu.VMEM((rows_per_w * 20,), jnp.float32),
            pltpu.VMEM((rows_per_w * 20,), jnp.float32),
            pltpu.VMEM((48 * _D,), jnp.float32),
            pltpu.VMEM((_D, 48), jnp.float32),
            pltpu.VMEM((32,), jnp.float32),
            pltpu.VMEM((32,), jnp.float32),
            pltpu.VMEM((48,), jnp.float32),
            pltpu.VMEM((rows_per_w,), jnp.float32),
            pltpu.VMEM((_L * 48,), jnp.float32),
            pltpu.VMEM((20, _L), jnp.float32),
            pltpu.VMEM((20, _L), jnp.float32),
            pltpu.VMEM((20, _L), jnp.float32),
            pltpu.VMEM((20, _L), jnp.float32),
            pltpu.VMEM((_L,), jnp.float32),
        ],
    )
    def k(stu_h, t1_h, t2_h, cw_h, w_h, out_h,
          stu_v, t1_v, t2_v, cw_v, cwT_v, w_v, wt_v, cne_v, an2_v,
          dv, p1T, p2T, w1b, w2b, acc_v):
        wid = lax.axis_index("s") * 2 + lax.axis_index("c")
        base = wid * rows_per_w
        pltpu.sync_copy(stu_h.at[pl.ds(base * _D, rows_per_w * _D)], stu_v)
        pltpu.sync_copy(t1_h.at[pl.ds(base * 20, rows_per_w * 20)], t1_v)
        pltpu.sync_copy(t2_h.at[pl.ds(base * 20, rows_per_w * 20)], t2_v)
        pltpu.sync_copy(cw_h, cw_v)
        pltpu.sync_copy(w_h, w_v)

        lanes = lax.iota(jnp.int32, _L)
        zero = jnp.zeros((_L,), jnp.float32)
        one = jnp.full((_L,), 1.0, jnp.float32)
        eps = jnp.float32(_EPS)

        for j in range(2):
            v = w_v[pl.ds(j * _L, _L)]
            s = one / (one + jnp.exp(-v))
            if j == 0:
                s = jnp.where(lanes == 0, zero, s)
            wt_v[pl.ds(j * _L, _L)] = s

        for j in range(3):
            cflat = (jnp.int32(j * _L) + lanes) * _D

            def cw_body(d, carry):
                sq, sm = carry
                col = plsc.load_gather(cw_v, [cflat + d])
                cwT_v[d, pl.ds(j * _L, _L)] = col
                return sq + col * col, sm + col

            sq, sm = lax.fori_loop(0, _D, cw_body, (zero, zero))
            cne_v[pl.ds(j * _L, _L)] = sq - (jnp.float32(2.0) * eps) * sm

        def an_chunk(ch, _):
            rflat = (ch * _L + lanes) * _D

            def an_body(d, carry):
                sq, sm = carry
                col = plsc.load_gather(stu_v, [rflat + d])
                return sq + col * col, sm + col

            sq, sm = lax.fori_loop(0, _D, an_body, (zero, zero), unroll=4)
            an2_v[pl.ds(ch * _L, _L)] = (
                sq + (jnp.float32(2.0) * eps) * sm
                + jnp.float32(_D) * eps * eps)
            return 0

        lax.fori_loop(0, n_chunk, an_chunk, 0)

        def chunk_body(ch, loss_acc):
            rbase = ch * _L
            ridx = rbase + lanes

            for sb in range(2):
                def mac(d, accs):
                    cwd = [cwT_v[d, pl.ds(j * _L, _L)] for j in range(3)]
                    out = []
                    for b in range(8):
                        row = rbase + (sb * 8 + b)
                        bv = plsc.load_gather(stu_v, [_bfull(row * _D) + d])
                        out.extend(accs[b * 3 + j] + bv * cwd[j]
                                   for j in range(3))
                    return tuple(out)

                dots = lax.fori_loop(0, _D, mac, (zero,) * 24, unroll=4)
                for b in range(8):
                    row = rbase + (sb * 8 + b)
                    an2 = plsc.load_gather(an2_v, [_bfull(row)])
                    for j in range(3):
                        d2 = an2 + cne_v[pl.ds(j * _L, _L)] \
                            - jnp.float32(2.0) * dots[b * 3 + j]
                        dv[pl.ds((sb * 8 + b) * 48 + j * _L, _L)] = _sqrt16(d2)

            rflat20 = ridx * 20

            def t_p(c, _):
                p1T[c, :] = plsc.load_gather(t1_v, [rflat20 + c])
                p2T[c, :] = plsc.load_gather(t2_v, [rflat20 + c])
                return 0

            lax.fori_loop(0, 20, t_p, 0)

            lanes48 = lanes * 48

            r2 = _ranks16([p2T[c, :] for c in range(20)])
            dpos1 = zero
            for c in range(20):
                w2b[c, :] = plsc.load_gather(wt_v, [r2[c].astype(jnp.int32)])
                dcol = plsc.load_gather(dv, [lanes48 + (20 + c)])
                dpos1 = dpos1 + jnp.where(r2[c] == 0.0, dcol, zero)

            r1 = _ranks16([p1T[c, :] for c in range(20)])
            dpos2 = zero
            for c in range(20):
                w1b[c, :] = plsc.load_gather(wt_v, [r1[c].astype(jnp.int32)])
                dcol = plsc.load_gather(dv, [lanes48 + c])
                dpos2 = dpos2 + jnp.where(r1[c] == 0.0, dcol, zero)

            m = jnp.full((_L,), _MARGIN, jnp.float32)
            for c in range(20):
                d1 = plsc.load_gather(dv, [lanes48 + c])
                d2c = plsc.load_gather(dv, [lanes48 + (20 + c)])
                loss_acc = loss_acc + w1b[c, :] * jnp.maximum(
                    dpos1 - d1 + m, zero)
                loss_acc = loss_acc + w2b[c, :] * jnp.maximum(
                    dpos2 - d2c + m, zero)
            return loss_acc

        acc = lax.fori_loop(0, n_chunk, chunk_body, zero)
        acc_v[...] = acc
        pltpu.sync_copy(acc_v, out_h.at[wid])

    return k(stu_f, t1_f, t2_f, cw_f, w32)


def _final_sum(parts, inv_b):
    def body(x_ref, o_ref):
        o_ref[...] = jnp.sum(x_ref[...], keepdims=True).reshape(1, 1) \
            * jnp.float32(inv_b)

    return pl.pallas_call(
        body, out_shape=jax.ShapeDtypeStruct((1, 1), jnp.float32)
    )(parts)


def kernel(stu_emb, t1_prob, t2_prob, classifier_weight, weights):
    b = stu_emb.shape[0]
    cw = lax.stop_gradient(classifier_weight)
    cw48 = jnp.zeros((48, _D), jnp.float32).at[:40].set(cw)
    w32 = jnp.zeros((32,), jnp.float32).at[:20].set(weights)
    parts = _sc_partials(
        b // _NW, stu_emb.reshape(-1), t1_prob.reshape(-1),
        t2_prob.reshape(-1), cw48.reshape(-1), w32)
    return _final_sum(parts, 1.0 / b)[0, 0]


# speedup vs baseline: 1.7937x; 1.2253x over previous
import functools

import jax
import jax.numpy as jnp
from jax import lax
from jax.experimental import pallas as pl
from jax.experimental.pallas import tpu as pltpu
from jax.experimental.pallas import tpu_sc as plsc

_MARGIN = 1.0
_EPS = 1e-6
_L = 16
_D = 64
_NW = 32


def _sqrt16(x):
    x = jnp.maximum(x, 0.0)
    xc = jnp.maximum(x, jnp.float32(1e-20))
    i = plsc.bitcast(xc, jnp.int32)
    i = jnp.int32(0x5F3759DF) - lax.shift_right_logical(i, 1)
    r = plsc.bitcast(i, jnp.float32)
    for _ in range(3):
        r = r * (jnp.float32(1.5) - jnp.float32(0.5) * xc * r * r)
    return x * r


def _ranks16(pv):
    one = jnp.full((_L,), 1.0, jnp.float32)
    zero = jnp.zeros((_L,), jnp.float32)
    rk = [zero] * 20
    for c in range(20):
        for cc in range(c + 1, 20):
            ge = pv[c] >= pv[cc]
            rk[cc] = rk[cc] + jnp.where(ge, one, zero)
            rk[c] = rk[c] + jnp.where(ge, zero, one)
    return rk


def _bfull(v):
    return jnp.full((_L,), v, jnp.int32)


_DN = lax.GatherDimensionNumbers(
    offset_dims=(), collapsed_slice_dims=(0,), start_index_map=(0,))


def _lane_bcast(v, lane):
    return lax.gather(v, _bfull(lane)[:, None], _DN, (1,),
                      mode=lax.GatherScatterMode.PROMISE_IN_BOUNDS)


def _sc_partials(rows_per_w, stu_f, t1_f, t2_f, cw_f, w32):
    n_chunk = rows_per_w // _L
    mesh = plsc.VectorSubcoreMesh(core_axis_name="c", subcore_axis_name="s")

    @functools.partial(
        pl.kernel,
        mesh=mesh,
        out_type=jax.ShapeDtypeStruct((_NW, _L), jnp.float32),
        compiler_params=pltpu.CompilerParams(needs_layout_passes=False),
        scratch_types=[
            pltpu.VMEM((rows_per_w * _D,), jnp.float32),
            pltpu.VMEM((rows_per_w * 20,), jnp.float32),
            pltpu.VMEM((rows_per_w * 20,), jnp.float32),
            pltpu.VMEM((48 * _D,), jnp.float32),
            pltpu.VMEM((_D, 48), jnp.float32),
            pltpu.VMEM((32,), jnp.float32),
            pltpu.VMEM((32,), jnp.float32),
            pltpu.VMEM((48,), jnp.float32),
            pltpu.VMEM((rows_per_w,), jnp.float32),
            pltpu.VMEM((_L * 48,), jnp.float32),
            pltpu.VMEM((20, _L), jnp.float32),
            pltpu.VMEM((20, _L), jnp.float32),
            pltpu.VMEM((20, _L), jnp.float32),
            pltpu.VMEM((20, _L), jnp.float32),
            pltpu.VMEM((_L,), jnp.float32),
        ],
    )
    def k(stu_h, t1_h, t2_h, cw_h, w_h, out_h,
          stu_v, t1_v, t2_v, cw_v, cwT_v, w_v, wt_v, cne_v, an2_v,
          dv, p1T, p2T, w1b, w2b, acc_v):
        wid = lax.axis_index("s") * 2 + lax.axis_index("c")
        base = wid * rows_per_w
        pltpu.sync_copy(stu_h.at[pl.ds(base * _D, rows_per_w * _D)], stu_v)
        pltpu.sync_copy(t1_h.at[pl.ds(base * 20, rows_per_w * 20)], t1_v)
        pltpu.sync_copy(t2_h.at[pl.ds(base * 20, rows_per_w * 20)], t2_v)
        pltpu.sync_copy(cw_h, cw_v)
        pltpu.sync_copy(w_h, w_v)

        lanes = lax.iota(jnp.int32, _L)
        zero = jnp.zeros((_L,), jnp.float32)
        one = jnp.full((_L,), 1.0, jnp.float32)
        eps = jnp.float32(_EPS)

        for j in range(2):
            v = w_v[pl.ds(j * _L, _L)]
            s = one / (one + jnp.exp(-v))
            if j == 0:
                s = jnp.where(lanes == 0, zero, s)
            wt_v[pl.ds(j * _L, _L)] = s

        for j in range(3):
            cflat = (jnp.int32(j * _L) + lanes) * _D

            def cw_body(d, carry):
                sq, sm = carry
                col = plsc.load_gather(cw_v, [cflat + d])
                cwT_v[d, pl.ds(j * _L, _L)] = col
                return sq + col * col, sm + col

            sq, sm = lax.fori_loop(0, _D, cw_body, (zero, zero))
            cne_v[pl.ds(j * _L, _L)] = sq - (jnp.float32(2.0) * eps) * sm

        def an_chunk(ch, _):
            rflat = (ch * _L + lanes) * _D

            def an_body(d, carry):
                sq, sm = carry
                col = plsc.load_gather(stu_v, [rflat + d])
                return sq + col * col, sm + col

            sq, sm = lax.fori_loop(0, _D, an_body, (zero, zero))
            an2_v[pl.ds(ch * _L, _L)] = (
                sq + (jnp.float32(2.0) * eps) * sm
                + jnp.float32(_D) * eps * eps)
            return 0

        lax.fori_loop(0, n_chunk, an_chunk, 0)

        def chunk_body(ch, loss_acc):
            rbase = ch * _L
            ridx = rbase + lanes

            for sb in range(2):
                bases = [_bfull((rbase + sb * 8 + b) * _D) for b in range(8)]

                def mac(d, accs):
                    bd = _bfull(d)
                    cwd = [cwT_v[d, pl.ds(j * _L, _L)] for j in range(3)]
                    out = []
                    for b in range(8):
                        bv = plsc.load_gather(stu_v, [bases[b] + bd])
                        out.extend(accs[b * 3 + j] + bv * cwd[j]
                                   for j in range(3))
                    return tuple(out)

                dots = lax.fori_loop(0, _D, mac, (zero,) * 24)
                for b in range(8):
                    row = rbase + (sb * 8 + b)
                    an2 = plsc.load_gather(an2_v, [_bfull(row)])
                    for j in range(3):
                        d2 = an2 + cne_v[pl.ds(j * _L, _L)] \
                            - jnp.float32(2.0) * dots[b * 3 + j]
                        dv[pl.ds((sb * 8 + b) * 48 + j * _L, _L)] = _sqrt16(d2)

            rflat20 = ridx * 20

            def t_p(c, _):
                p1T[c, :] = plsc.load_gather(t1_v, [rflat20 + c])
                p2T[c, :] = plsc.load_gather(t2_v, [rflat20 + c])
                return 0

            lax.fori_loop(0, 20, t_p, 0)

            lanes48 = lanes * 48

            r2 = _ranks16([p2T[c, :] for c in range(20)])
            dpos1 = zero
            for c in range(20):
                w2b[c, :] = plsc.load_gather(wt_v, [r2[c].astype(jnp.int32)])
                dcol = plsc.load_gather(dv, [lanes48 + (20 + c)])
                dpos1 = dpos1 + jnp.where(r2[c] == 0.0, dcol, zero)

            r1 = _ranks16([p1T[c, :] for c in range(20)])
            dpos2 = zero
            for c in range(20):
                w1b[c, :] = plsc.load_gather(wt_v, [r1[c].astype(jnp.int32)])
                dcol = plsc.load_gather(dv, [lanes48 + c])
                dpos2 = dpos2 + jnp.where(r1[c] == 0.0, dcol, zero)

            m = jnp.full((_L,), _MARGIN, jnp.float32)
            for c in range(20):
                d1 = plsc.load_gather(dv, [lanes48 + c])
                d2c = plsc.load_gather(dv, [lanes48 + (20 + c)])
                loss_acc = loss_acc + w1b[c, :] * jnp.maximum(
                    dpos1 - d1 + m, zero)
                loss_acc = loss_acc + w2b[c, :] * jnp.maximum(
                    dpos2 - d2c + m, zero)
            return loss_acc

        acc = lax.fori_loop(0, n_chunk, chunk_body, zero)
        acc_v[...] = acc
        pltpu.sync_copy(acc_v, out_h.at[wid])

    return k(stu_f, t1_f, t2_f, cw_f, w32)


def _final_sum(parts, inv_b):
    def body(x_ref, o_ref):
        o_ref[...] = jnp.sum(x_ref[...], keepdims=True).reshape(1, 1) \
            * jnp.float32(inv_b)

    return pl.pallas_call(
        body, out_shape=jax.ShapeDtypeStruct((1, 1), jnp.float32)
    )(parts)


def kernel(stu_emb, t1_prob, t2_prob, classifier_weight, weights):
    b = stu_emb.shape[0]
    cw = lax.stop_gradient(classifier_weight)
    cw48 = jnp.zeros((48, _D), jnp.float32).at[:40].set(cw)
    w32 = jnp.zeros((32,), jnp.float32).at[:20].set(weights)
    parts = _sc_partials(
        b // _NW, stu_emb.reshape(-1), t1_prob.reshape(-1),
        t2_prob.reshape(-1), cw48.reshape(-1), w32)
    return _final_sum(parts, 1.0 / b)[0, 0]


# speedup vs baseline: 3.3075x; 1.8439x over previous
import functools

import jax
import jax.numpy as jnp
from jax import lax
from jax.experimental import pallas as pl
from jax.experimental.pallas import tpu as pltpu
from jax.experimental.pallas import tpu_sc as plsc

_MARGIN = 1.0
_EPS = 1e-6
_L = 16
_D = 64
_NW = 32
_NC = 48


def _ranks16(pv):
    one = jnp.full((_L,), 1.0, jnp.float32)
    zero = jnp.zeros((_L,), jnp.float32)
    rk = [zero] * 20
    for c in range(20):
        for cc in range(c + 1, 20):
            ge = pv[c] >= pv[cc]
            rk[cc] = rk[cc] + jnp.where(ge, one, zero)
            rk[c] = rk[c] + jnp.where(ge, zero, one)
    return rk


def _bfull(v):
    return jnp.full((_L,), v, jnp.int32)


def _dist_tc(stu, cwT):
    b = stu.shape[0]
    rb = 1024

    def body(x_ref, w_ref, o_ref):
        a = x_ref[...] + jnp.float32(_EPS)
        w = w_ref[...]
        an2 = jnp.sum(a * a, axis=1, keepdims=True)
        cn = jnp.sum(w * w, axis=0, keepdims=True)
        dot = jnp.dot(a, w, preferred_element_type=jnp.float32)
        d2 = an2 + cn - jnp.float32(2.0) * dot
        o_ref[...] = jnp.sqrt(jnp.maximum(d2, 0.0))

    return pl.pallas_call(
        body,
        grid=(b // rb,),
        in_specs=[
            pl.BlockSpec((rb, _D), lambda i: (i, 0)),
            pl.BlockSpec((_D, _NC), lambda i: (0, 0)),
        ],
        out_specs=pl.BlockSpec((rb, _NC), lambda i: (i, 0)),
        out_shape=jax.ShapeDtypeStruct((b, _NC), jnp.float32),
    )(stu, cwT)


def _sc_partials(rows_per_w, dist_f, t1_f, t2_f, w32):
    n_chunk = rows_per_w // _L
    mesh = plsc.VectorSubcoreMesh(core_axis_name="c", subcore_axis_name="s")

    @functools.partial(
        pl.kernel,
        mesh=mesh,
        out_type=jax.ShapeDtypeStruct((_NW, _L), jnp.float32),
        compiler_params=pltpu.CompilerParams(needs_layout_passes=False),
        scratch_types=[
            pltpu.VMEM((rows_per_w * _NC,), jnp.float32),
            pltpu.VMEM((rows_per_w * 20,), jnp.float32),
            pltpu.VMEM((rows_per_w * 20,), jnp.float32),
            pltpu.VMEM((32,), jnp.float32),
            pltpu.VMEM((32,), jnp.float32),
            pltpu.VMEM((20, _L), jnp.float32),
            pltpu.VMEM((20, _L), jnp.float32),
            pltpu.VMEM((20, _L), jnp.float32),
            pltpu.VMEM((20, _L), jnp.float32),
            pltpu.VMEM((_L,), jnp.float32),
        ],
    )
    def k(dist_h, t1_h, t2_h, w_h, out_h,
          dist_v, t1_v, t2_v, w_v, wt_v, p1T, p2T, w1b, w2b, acc_v):
        wid = lax.axis_index("s") * 2 + lax.axis_index("c")
        base = wid * rows_per_w
        pltpu.sync_copy(dist_h.at[pl.ds(base * _NC, rows_per_w * _NC)], dist_v)
        pltpu.sync_copy(t1_h.at[pl.ds(base * 20, rows_per_w * 20)], t1_v)
        pltpu.sync_copy(t2_h.at[pl.ds(base * 20, rows_per_w * 20)], t2_v)
        pltpu.sync_copy(w_h, w_v)

        lanes = lax.iota(jnp.int32, _L)
        zero = jnp.zeros((_L,), jnp.float32)
        one = jnp.full((_L,), 1.0, jnp.float32)

        for j in range(2):
            v = w_v[pl.ds(j * _L, _L)]
            s = one / (one + jnp.exp(-v))
            if j == 0:
                s = jnp.where(lanes == 0, zero, s)
            wt_v[pl.ds(j * _L, _L)] = s

        def chunk_body(ch, loss_acc):
            ridx = ch * _L + lanes
            rflat20 = ridx * 20
            rflat48 = ridx * _NC

            def t_p(c, _):
                p1T[c, :] = plsc.load_gather(t1_v, [rflat20 + c])
                p2T[c, :] = plsc.load_gather(t2_v, [rflat20 + c])
                return 0

            lax.fori_loop(0, 20, t_p, 0)

            r2 = _ranks16([p2T[c, :] for c in range(20)])
            dpos1 = zero
            for c in range(20):
                w2b[c, :] = plsc.load_gather(wt_v, [r2[c].astype(jnp.int32)])
                dcol = plsc.load_gather(dist_v, [rflat48 + (20 + c)])
                dpos1 = dpos1 + jnp.where(r2[c] == 0.0, dcol, zero)

            r1 = _ranks16([p1T[c, :] for c in range(20)])
            dpos2 = zero
            for c in range(20):
                w1b[c, :] = plsc.load_gather(wt_v, [r1[c].astype(jnp.int32)])
                dcol = plsc.load_gather(dist_v, [rflat48 + c])
                dpos2 = dpos2 + jnp.where(r1[c] == 0.0, dcol, zero)

            m = jnp.full((_L,), _MARGIN, jnp.float32)
            for c in range(20):
                d1 = plsc.load_gather(dist_v, [rflat48 + c])
                d2c = plsc.load_gather(dist_v, [rflat48 + (20 + c)])
                loss_acc = loss_acc + w1b[c, :] * jnp.maximum(
                    dpos1 - d1 + m, zero)
                loss_acc = loss_acc + w2b[c, :] * jnp.maximum(
                    dpos2 - d2c + m, zero)
            return loss_acc

        acc = lax.fori_loop(0, n_chunk, chunk_body, zero)
        acc_v[...] = acc
        pltpu.sync_copy(acc_v, out_h.at[wid])

    return k(dist_f, t1_f, t2_f, w32)


def _final_sum(parts, inv_b):
    def body(x_ref, o_ref):
        o_ref[...] = jnp.sum(x_ref[...], keepdims=True).reshape(1, 1) \
            * jnp.float32(inv_b)

    return pl.pallas_call(
        body, out_shape=jax.ShapeDtypeStruct((1, 1), jnp.float32)
    )(parts)


def kernel(stu_emb, t1_prob, t2_prob, classifier_weight, weights):
    b = stu_emb.shape[0]
    cw = lax.stop_gradient(classifier_weight)
    cwT = jnp.zeros((_D, _NC), jnp.float32).at[:, :40].set(cw.T)
    w32 = jnp.zeros((32,), jnp.float32).at[:20].set(weights)
    dist = _dist_tc(stu_emb, cwT)
    parts = _sc_partials(
        b // _NW, dist.reshape(-1), t1_prob.reshape(-1),
        t2_prob.reshape(-1), w32)
    return _final_sum(parts, 1.0 / b)[0, 0]


# speedup vs baseline: 3.9190x; 1.1849x over previous
import functools

import jax
import jax.numpy as jnp
from jax import lax
from jax.experimental import pallas as pl
from jax.experimental.pallas import tpu as pltpu
from jax.experimental.pallas import tpu_sc as plsc

_MARGIN = 1.0
_EPS = 1e-6
_L = 16
_D = 64
_NW = 32
_NC = 48


def _ranks16(pv):
    one = jnp.full((_L,), 1.0, jnp.float32)
    zero = jnp.zeros((_L,), jnp.float32)
    rk = [zero] * 20
    for c in range(20):
        for cc in range(c + 1, 20):
            ge = pv[c] >= pv[cc]
            rk[cc] = rk[cc] + jnp.where(ge, one, zero)
            rk[c] = rk[c] + jnp.where(ge, zero, one)
    return rk


def _bfull(v):
    return jnp.full((_L,), v, jnp.int32)


def _dist_tc(stu, cw48):
    b = stu.shape[0]
    rb = 1024

    def body(w_ref, x_ref, o_ref):
        a = x_ref[...] + jnp.float32(_EPS)
        w = w_ref[...]
        an2 = jnp.sum(a * a, axis=1)[None, :]
        cn = jnp.sum(w * w, axis=1, keepdims=True)
        dot = lax.dot_general(w, a, (((1,), (1,)), ((), ())),
                              preferred_element_type=jnp.float32)
        d2 = cn + an2 - jnp.float32(2.0) * dot
        o_ref[...] = jnp.sqrt(jnp.maximum(d2, 0.0))

    return pl.pallas_call(
        body,
        grid=(b // rb,),
        in_specs=[
            pl.BlockSpec((_NC, _D), lambda i: (0, 0)),
            pl.BlockSpec((rb, _D), lambda i: (i, 0)),
        ],
        out_specs=pl.BlockSpec((_NC, rb), lambda i: (0, i)),
        out_shape=jax.ShapeDtypeStruct((_NC, b), jnp.float32),
    )(cw48, stu)


def _sc_partials(rows_per_w, dist_f, t1_f, t2_f, w32):
    n_chunk = rows_per_w // _L
    mesh = plsc.VectorSubcoreMesh(core_axis_name="c", subcore_axis_name="s")

    @functools.partial(
        pl.kernel,
        mesh=mesh,
        out_type=jax.ShapeDtypeStruct((_NW, _L), jnp.float32),
        compiler_params=pltpu.CompilerParams(needs_layout_passes=False),
        scratch_types=[
            pltpu.VMEM((_NC, rows_per_w), jnp.float32),
            pltpu.VMEM((rows_per_w * 20,), jnp.float32),
            pltpu.VMEM((rows_per_w * 20,), jnp.float32),
            pltpu.VMEM((32,), jnp.float32),
            pltpu.VMEM((32,), jnp.float32),
            pltpu.VMEM((20, _L), jnp.float32),
            pltpu.VMEM((20, _L), jnp.float32),
            pltpu.VMEM((20, _L), jnp.float32),
            pltpu.VMEM((20, _L), jnp.float32),
            pltpu.VMEM((_L,), jnp.float32),
        ],
    )
    def k(dist_h, t1_h, t2_h, w_h, out_h,
          dist_v, t1_v, t2_v, w_v, wt_v, p1T, p2T, w1b, w2b, acc_v):
        wid = lax.axis_index("s") * 2 + lax.axis_index("c")
        base = wid * rows_per_w
        pltpu.sync_copy(dist_h.at[:, pl.ds(base, rows_per_w)], dist_v)
        pltpu.sync_copy(t1_h.at[pl.ds(base * 20, rows_per_w * 20)], t1_v)
        pltpu.sync_copy(t2_h.at[pl.ds(base * 20, rows_per_w * 20)], t2_v)
        pltpu.sync_copy(w_h, w_v)

        lanes = lax.iota(jnp.int32, _L)
        zero = jnp.zeros((_L,), jnp.float32)
        one = jnp.full((_L,), 1.0, jnp.float32)

        for j in range(2):
            v = w_v[pl.ds(j * _L, _L)]
            s = one / (one + jnp.exp(-v))
            if j == 0:
                s = jnp.where(lanes == 0, zero, s)
            wt_v[pl.ds(j * _L, _L)] = s

        def chunk_body(ch, loss_acc):
            rbase = ch * _L
            ridx = rbase + lanes
            rflat20 = ridx * 20

            def t_p(c, _):
                p1T[c, :] = plsc.load_gather(t1_v, [rflat20 + c])
                p2T[c, :] = plsc.load_gather(t2_v, [rflat20 + c])
                return 0

            lax.fori_loop(0, 20, t_p, 0)

            r2 = _ranks16([p2T[c, :] for c in range(20)])
            dpos1 = zero
            for c in range(20):
                w2b[c, :] = plsc.load_gather(wt_v, [r2[c].astype(jnp.int32)])
                dcol = dist_v[20 + c, pl.ds(rbase, _L)]
                dpos1 = dpos1 + jnp.where(r2[c] == 0.0, dcol, zero)

            r1 = _ranks16([p1T[c, :] for c in range(20)])
            dpos2 = zero
            for c in range(20):
                w1b[c, :] = plsc.load_gather(wt_v, [r1[c].astype(jnp.int32)])
                dcol = dist_v[c, pl.ds(rbase, _L)]
                dpos2 = dpos2 + jnp.where(r1[c] == 0.0, dcol, zero)

            m = jnp.full((_L,), _MARGIN, jnp.float32)
            for c in range(20):
                d1 = dist_v[c, pl.ds(rbase, _L)]
                d2c = dist_v[20 + c, pl.ds(rbase, _L)]
                loss_acc = loss_acc + w1b[c, :] * jnp.maximum(
                    dpos1 - d1 + m, zero)
                loss_acc = loss_acc + w2b[c, :] * jnp.maximum(
                    dpos2 - d2c + m, zero)
            return loss_acc

        acc = lax.fori_loop(0, n_chunk, chunk_body, zero)
        acc_v[...] = acc
        pltpu.sync_copy(acc_v, out_h.at[wid])

    return k(dist_f, t1_f, t2_f, w32)


def _final_sum(parts, inv_b):
    def body(x_ref, o_ref):
        o_ref[...] = jnp.sum(x_ref[...], keepdims=True).reshape(1, 1) \
            * jnp.float32(inv_b)

    return pl.pallas_call(
        body, out_shape=jax.ShapeDtypeStruct((1, 1), jnp.float32)
    )(parts)


def kernel(stu_emb, t1_prob, t2_prob, classifier_weight, weights):
    b = stu_emb.shape[0]
    cw = lax.stop_gradient(classifier_weight)
    cw48 = jnp.zeros((_NC, _D), jnp.float32).at[:40].set(cw)
    w32 = jnp.zeros((32,), jnp.float32).at[:20].set(weights)
    dist = _dist_tc(stu_emb, cw48)
    parts = _sc_partials(
        b // _NW, dist, t1_prob.reshape(-1), t2_prob.reshape(-1), w32)
    return _final_sum(parts, 1.0 / b)[0, 0]


# speedup vs baseline: 4.0844x; 1.0422x over previous
import functools

import jax
import jax.numpy as jnp
from jax import lax
from jax.experimental import pallas as pl
from jax.experimental.pallas import tpu as pltpu
from jax.experimental.pallas import tpu_sc as plsc

_MARGIN = 1.0
_EPS = 1e-6
_L = 16
_D = 64
_NW = 32
_NC = 48


def _ranks16(pv):
    one = jnp.full((_L,), 1.0, jnp.float32)
    zero = jnp.zeros((_L,), jnp.float32)
    rk = [zero] * 20
    for c in range(20):
        for cc in range(c + 1, 20):
            ge = pv[c] >= pv[cc]
            rk[cc] = rk[cc] + jnp.where(ge, one, zero)
            rk[c] = rk[c] + jnp.where(ge, zero, one)
    return rk


def _bfull(v):
    return jnp.full((_L,), v, jnp.int32)


def _dist_tc(stu, cw48):
    b = stu.shape[0]
    rb = 1024

    def body(w_ref, x_ref, o_ref):
        a = x_ref[...] + jnp.float32(_EPS)
        w = w_ref[...]
        an2 = jnp.sum(a * a, axis=1)[None, :]
        cn = jnp.sum(w * w, axis=1, keepdims=True)
        dot = lax.dot_general(w, a, (((1,), (1,)), ((), ())),
                              preferred_element_type=jnp.float32)
        d2 = cn + an2 - jnp.float32(2.0) * dot
        o_ref[...] = jnp.sqrt(jnp.maximum(d2, 0.0))

    return pl.pallas_call(
        body,
        grid=(b // rb,),
        in_specs=[
            pl.BlockSpec((_NC, _D), lambda i: (0, 0)),
            pl.BlockSpec((rb, _D), lambda i: (i, 0)),
        ],
        out_specs=pl.BlockSpec((_NC, rb), lambda i: (0, i)),
        out_shape=jax.ShapeDtypeStruct((_NC, b), jnp.float32),
    )(cw48, stu)


def _sc_partials(rows_per_w, dist_f, t1_f, t2_f, w32):
    n_chunk = rows_per_w // _L
    mesh = plsc.VectorSubcoreMesh(core_axis_name="c", subcore_axis_name="s")

    @functools.partial(
        pl.kernel,
        mesh=mesh,
        out_type=jax.ShapeDtypeStruct((_NW, _L), jnp.float32),
        compiler_params=pltpu.CompilerParams(needs_layout_passes=False),
        scratch_types=[
            pltpu.VMEM((_NC, rows_per_w), jnp.float32),
            pltpu.VMEM((rows_per_w * 20,), jnp.float32),
            pltpu.VMEM((rows_per_w * 20,), jnp.float32),
            pltpu.VMEM((32,), jnp.float32),
            pltpu.VMEM((32,), jnp.float32),
            pltpu.VMEM((20, _L), jnp.float32),
            pltpu.VMEM((20, _L), jnp.float32),
            pltpu.VMEM((_L,), jnp.float32),
        ],
    )
    def k(dist_h, t1_h, t2_h, w_h, out_h,
          dist_v, t1_v, t2_v, w_v, wt_v, w1b, w2b, acc_v):
        wid = lax.axis_index("s") * 2 + lax.axis_index("c")
        base = wid * rows_per_w
        pltpu.sync_copy(dist_h.at[:, pl.ds(base, rows_per_w)], dist_v)
        pltpu.sync_copy(t1_h.at[pl.ds(base * 20, rows_per_w * 20)], t1_v)
        pltpu.sync_copy(t2_h.at[pl.ds(base * 20, rows_per_w * 20)], t2_v)
        pltpu.sync_copy(w_h, w_v)

        lanes = lax.iota(jnp.int32, _L)
        zero = jnp.zeros((_L,), jnp.float32)
        one = jnp.full((_L,), 1.0, jnp.float32)

        for j in range(2):
            v = w_v[pl.ds(j * _L, _L)]
            s = one / (one + jnp.exp(-v))
            if j == 0:
                s = jnp.where(lanes == 0, zero, s)
            wt_v[pl.ds(j * _L, _L)] = s

        def chunk_body(ch, loss_acc):
            rbase = ch * _L
            ridx = rbase + lanes
            rflat20 = ridx * 20


            p2v = [plsc.load_gather(t2_v, [rflat20 + c]) for c in range(20)]
            r2 = _ranks16(p2v)
            dpos1 = zero
            for c in range(20):
                w2b[c, :] = plsc.load_gather(wt_v, [r2[c].astype(jnp.int32)])
                dcol = dist_v[20 + c, pl.ds(rbase, _L)]
                dpos1 = dpos1 + jnp.where(r2[c] == 0.0, dcol, zero)

            p1v = [plsc.load_gather(t1_v, [rflat20 + c]) for c in range(20)]
            r1 = _ranks16(p1v)
            dpos2 = zero
            for c in range(20):
                w1b[c, :] = plsc.load_gather(wt_v, [r1[c].astype(jnp.int32)])
                dcol = dist_v[c, pl.ds(rbase, _L)]
                dpos2 = dpos2 + jnp.where(r1[c] == 0.0, dcol, zero)

            m = jnp.full((_L,), _MARGIN, jnp.float32)
            for c in range(20):
                d1 = dist_v[c, pl.ds(rbase, _L)]
                d2c = dist_v[20 + c, pl.ds(rbase, _L)]
                loss_acc = loss_acc + w1b[c, :] * jnp.maximum(
                    dpos1 - d1 + m, zero)
                loss_acc = loss_acc + w2b[c, :] * jnp.maximum(
                    dpos2 - d2c + m, zero)
            return loss_acc

        acc = lax.fori_loop(0, n_chunk, chunk_body, zero)
        acc_v[...] = acc
        pltpu.sync_copy(acc_v, out_h.at[wid])

    return k(dist_f, t1_f, t2_f, w32)


def _final_sum(parts, inv_b):
    def body(x_ref, o_ref):
        o_ref[...] = jnp.sum(x_ref[...], keepdims=True).reshape(1, 1) \
            * jnp.float32(inv_b)

    return pl.pallas_call(
        body, out_shape=jax.ShapeDtypeStruct((1, 1), jnp.float32)
    )(parts)


def kernel(stu_emb, t1_prob, t2_prob, classifier_weight, weights):
    b = stu_emb.shape[0]
    cw = lax.stop_gradient(classifier_weight)
    cw48 = jnp.zeros((_NC, _D), jnp.float32).at[:40].set(cw)
    w32 = jnp.zeros((32,), jnp.float32).at[:20].set(weights)
    dist = _dist_tc(stu_emb, cw48)
    parts = _sc_partials(
        b // _NW, dist, t1_prob.reshape(-1), t2_prob.reshape(-1), w32)
    return _final_sum(parts, 1.0 / b)[0, 0]


# speedup vs baseline: 4.7703x; 1.1679x over previous
import functools

import jax
import jax.numpy as jnp
from jax import lax
from jax.experimental import pallas as pl
from jax.experimental.pallas import tpu as pltpu
from jax.experimental.pallas import tpu_sc as plsc

_MARGIN = 1.0
_EPS = 1e-6
_L = 16
_D = 64
_NW = 32
_NC = 48


def _ranks16(pv):
    one = jnp.full((_L,), 1.0, jnp.float32)
    zero = jnp.zeros((_L,), jnp.float32)
    rk = [zero] * 20
    for c in range(20):
        for cc in range(c + 1, 20):
            ge = pv[c] >= pv[cc]
            rk[cc] = rk[cc] + jnp.where(ge, one, zero)
            rk[c] = rk[c] + jnp.where(ge, zero, one)
    return rk


def _bfull(v):
    return jnp.full((_L,), v, jnp.int32)


def _dist_tc(stu, cw48):
    b = stu.shape[1]
    rb = 4096

    def body(w_ref, x_ref, o_ref):
        a = x_ref[...] + jnp.float32(_EPS)
        w = w_ref[...]
        an2 = jnp.sum(a * a, axis=0)[None, :]
        cn = jnp.sum(w * w, axis=1, keepdims=True)
        dot = jnp.dot(w, a, preferred_element_type=jnp.float32)
        d2 = cn + an2 - jnp.float32(2.0) * dot
        o_ref[...] = jnp.sqrt(jnp.maximum(d2, 0.0))

    return pl.pallas_call(
        body,
        grid=(b // rb,),
        in_specs=[
            pl.BlockSpec((_NC, _D), lambda i: (0, 0)),
            pl.BlockSpec((_D, rb), lambda i: (0, i)),
        ],
        out_specs=pl.BlockSpec((_NC, rb), lambda i: (0, i)),
        out_shape=jax.ShapeDtypeStruct((_NC, b), jnp.float32),
    )(cw48, stu)


def _sc_partials(rows_per_w, dist_f, t1_f, t2_f, w32):
    n_chunk = rows_per_w // _L
    mesh = plsc.VectorSubcoreMesh(core_axis_name="c", subcore_axis_name="s")

    @functools.partial(
        pl.kernel,
        mesh=mesh,
        out_type=jax.ShapeDtypeStruct((_NW, _L), jnp.float32),
        compiler_params=pltpu.CompilerParams(needs_layout_passes=False),
        scratch_types=[
            pltpu.VMEM((_NC, rows_per_w), jnp.float32),
            pltpu.VMEM((rows_per_w * 20,), jnp.float32),
            pltpu.VMEM((rows_per_w * 20,), jnp.float32),
            pltpu.VMEM((32,), jnp.float32),
            pltpu.VMEM((32,), jnp.float32),
            pltpu.VMEM((20, _L), jnp.float32),
            pltpu.VMEM((20, _L), jnp.float32),
            pltpu.VMEM((_L,), jnp.float32),
        ],
    )
    def k(dist_h, t1_h, t2_h, w_h, out_h,
          dist_v, t1_v, t2_v, w_v, wt_v, w1b, w2b, acc_v):
        wid = lax.axis_index("s") * 2 + lax.axis_index("c")
        base = wid * rows_per_w
        pltpu.sync_copy(dist_h.at[:, pl.ds(base, rows_per_w)], dist_v)
        pltpu.sync_copy(t1_h.at[pl.ds(base * 20, rows_per_w * 20)], t1_v)
        pltpu.sync_copy(t2_h.at[pl.ds(base * 20, rows_per_w * 20)], t2_v)
        pltpu.sync_copy(w_h, w_v)

        lanes = lax.iota(jnp.int32, _L)
        zero = jnp.zeros((_L,), jnp.float32)
        one = jnp.full((_L,), 1.0, jnp.float32)

        for j in range(2):
            v = w_v[pl.ds(j * _L, _L)]
            s = one / (one + jnp.exp(-v))
            if j == 0:
                s = jnp.where(lanes == 0, zero, s)
            wt_v[pl.ds(j * _L, _L)] = s

        def chunk_body(ch, loss_acc):
            rbase = ch * _L
            ridx = rbase + lanes
            rflat20 = ridx * 20


            p2v = [plsc.load_gather(t2_v, [rflat20 + c]) for c in range(20)]
            r2 = _ranks16(p2v)
            dpos1 = zero
            for c in range(20):
                w2b[c, :] = plsc.load_gather(wt_v, [r2[c].astype(jnp.int32)])
                dcol = dist_v[20 + c, pl.ds(rbase, _L)]
                dpos1 = dpos1 + jnp.where(r2[c] == 0.0, dcol, zero)

            p1v = [plsc.load_gather(t1_v, [rflat20 + c]) for c in range(20)]
            r1 = _ranks16(p1v)
            dpos2 = zero
            for c in range(20):
                w1b[c, :] = plsc.load_gather(wt_v, [r1[c].astype(jnp.int32)])
                dcol = dist_v[c, pl.ds(rbase, _L)]
                dpos2 = dpos2 + jnp.where(r1[c] == 0.0, dcol, zero)

            m = jnp.full((_L,), _MARGIN, jnp.float32)
            for c in range(20):
                d1 = dist_v[c, pl.ds(rbase, _L)]
                d2c = dist_v[20 + c, pl.ds(rbase, _L)]
                loss_acc = loss_acc + w1b[c, :] * jnp.maximum(
                    dpos1 - d1 + m, zero)
                loss_acc = loss_acc + w2b[c, :] * jnp.maximum(
                    dpos2 - d2c + m, zero)
            return loss_acc

        acc = lax.fori_loop(0, n_chunk, chunk_body, zero)
        acc_v[...] = acc
        pltpu.sync_copy(acc_v, out_h.at[wid])

    return k(dist_f, t1_f, t2_f, w32)


def _final_sum(parts, inv_b):
    def body(x_ref, o_ref):
        o_ref[...] = jnp.sum(x_ref[...], keepdims=True).reshape(1, 1) \
            * jnp.float32(inv_b)

    return pl.pallas_call(
        body, out_shape=jax.ShapeDtypeStruct((1, 1), jnp.float32)
    )(parts)


def kernel(stu_emb, t1_prob, t2_prob, classifier_weight, weights):
    b = stu_emb.shape[0]
    cw = lax.stop_gradient(classifier_weight)
    cw48 = jnp.zeros((_NC, _D), jnp.float32).at[:40].set(cw)
    w32 = jnp.zeros((32,), jnp.float32).at[:20].set(weights)
    dist = _dist_tc(stu_emb.T, cw48)
    parts = _sc_partials(
        b // _NW, dist, t1_prob.reshape(-1), t2_prob.reshape(-1), w32)
    return _final_sum(parts, 1.0 / b)[0, 0]


# speedup vs baseline: 4.7966x; 1.0055x over previous
import functools

import jax
import jax.numpy as jnp
from jax import lax
from jax.experimental import pallas as pl
from jax.experimental.pallas import tpu as pltpu
from jax.experimental.pallas import tpu_sc as plsc

_MARGIN = 1.0
_EPS = 1e-6
_L = 16
_D = 64
_NW = 32
_NC = 48


def _ranks16(pv):
    rk = [jnp.full((_L,), float(19 - c), jnp.float32) for c in range(20)]
    for c in range(20):
        for cc in range(c + 1, 20):
            t = (pv[c] >= pv[cc]).astype(jnp.float32)
            rk[cc] = rk[cc] + t
            rk[c] = rk[c] - t
    return rk


def _bfull(v):
    return jnp.full((_L,), v, jnp.int32)


def _dist_tc(stu, cw48):
    b = stu.shape[1]
    rb = 4096

    def body(w_ref, x_ref, o_ref):
        a = x_ref[...] + jnp.float32(_EPS)
        w = w_ref[...]
        an2 = jnp.sum(a * a, axis=0)[None, :]
        cn = jnp.sum(w * w, axis=1, keepdims=True)
        dot = jnp.dot(w, a, preferred_element_type=jnp.float32)
        d2 = cn + an2 - jnp.float32(2.0) * dot
        o_ref[...] = jnp.sqrt(jnp.maximum(d2, 0.0))

    return pl.pallas_call(
        body,
        grid=(b // rb,),
        in_specs=[
            pl.BlockSpec((_NC, _D), lambda i: (0, 0)),
            pl.BlockSpec((_D, rb), lambda i: (0, i)),
        ],
        out_specs=pl.BlockSpec((_NC, rb), lambda i: (0, i)),
        out_shape=jax.ShapeDtypeStruct((_NC, b), jnp.float32),
    )(cw48, stu)


def _sc_partials(rows_per_w, dist_f, t1_f, t2_f, w32):
    n_chunk = rows_per_w // _L
    mesh = plsc.VectorSubcoreMesh(core_axis_name="c", subcore_axis_name="s")

    @functools.partial(
        pl.kernel,
        mesh=mesh,
        out_type=jax.ShapeDtypeStruct((_NW, _L), jnp.float32),
        compiler_params=pltpu.CompilerParams(needs_layout_passes=False),
        scratch_types=[
            pltpu.VMEM((_NC, rows_per_w), jnp.float32),
            pltpu.VMEM((rows_per_w * 20,), jnp.float32),
            pltpu.VMEM((rows_per_w * 20,), jnp.float32),
            pltpu.VMEM((32,), jnp.float32),
            pltpu.VMEM((32,), jnp.float32),
            pltpu.VMEM((20, _L), jnp.float32),
            pltpu.VMEM((20, _L), jnp.float32),
            pltpu.VMEM((_L,), jnp.float32),
        ],
    )
    def k(dist_h, t1_h, t2_h, w_h, out_h,
          dist_v, t1_v, t2_v, w_v, wt_v, w1b, w2b, acc_v):
        wid = lax.axis_index("s") * 2 + lax.axis_index("c")
        base = wid * rows_per_w
        pltpu.sync_copy(dist_h.at[:, pl.ds(base, rows_per_w)], dist_v)
        pltpu.sync_copy(t1_h.at[pl.ds(base * 20, rows_per_w * 20)], t1_v)
        pltpu.sync_copy(t2_h.at[pl.ds(base * 20, rows_per_w * 20)], t2_v)
        pltpu.sync_copy(w_h, w_v)

        lanes = lax.iota(jnp.int32, _L)
        zero = jnp.zeros((_L,), jnp.float32)
        one = jnp.full((_L,), 1.0, jnp.float32)

        for j in range(2):
            v = w_v[pl.ds(j * _L, _L)]
            s = one / (one + jnp.exp(-v))
            if j == 0:
                s = jnp.where(lanes == 0, zero, s)
            wt_v[pl.ds(j * _L, _L)] = s

        def chunk_body(ch, loss_acc):
            rbase = ch * _L
            ridx = rbase + lanes
            rflat20 = ridx * 20


            p2v = [plsc.load_gather(t2_v, [rflat20 + c]) for c in range(20)]
            r2 = _ranks16(p2v)
            dpos1 = zero
            for c in range(20):
                w2b[c, :] = plsc.load_gather(wt_v, [r2[c].astype(jnp.int32)])
                dcol = dist_v[20 + c, pl.ds(rbase, _L)]
                dpos1 = dpos1 + jnp.where(r2[c] == 0.0, dcol, zero)

            p1v = [plsc.load_gather(t1_v, [rflat20 + c]) for c in range(20)]
            r1 = _ranks16(p1v)
            dpos2 = zero
            for c in range(20):
                w1b[c, :] = plsc.load_gather(wt_v, [r1[c].astype(jnp.int32)])
                dcol = dist_v[c, pl.ds(rbase, _L)]
                dpos2 = dpos2 + jnp.where(r1[c] == 0.0, dcol, zero)

            m = jnp.full((_L,), _MARGIN, jnp.float32)
            for c in range(20):
                d1 = dist_v[c, pl.ds(rbase, _L)]
                d2c = dist_v[20 + c, pl.ds(rbase, _L)]
                loss_acc = loss_acc + w1b[c, :] * jnp.maximum(
                    dpos1 - d1 + m, zero)
                loss_acc = loss_acc + w2b[c, :] * jnp.maximum(
                    dpos2 - d2c + m, zero)
            return loss_acc

        acc = lax.fori_loop(0, n_chunk, chunk_body, zero)
        acc_v[...] = acc
        pltpu.sync_copy(acc_v, out_h.at[wid])

    return k(dist_f, t1_f, t2_f, w32)


def _final_sum(parts, inv_b):
    def body(x_ref, o_ref):
        o_ref[...] = jnp.sum(x_ref[...], keepdims=True).reshape(1, 1) \
            * jnp.float32(inv_b)

    return pl.pallas_call(
        body, out_shape=jax.ShapeDtypeStruct((1, 1), jnp.float32)
    )(parts)


def kernel(stu_emb, t1_prob, t2_prob, classifier_weight, weights):
    b = stu_emb.shape[0]
    cw = lax.stop_gradient(classifier_weight)
    cw48 = jnp.zeros((_NC, _D), jnp.float32).at[:40].set(cw)
    w32 = jnp.zeros((32,), jnp.float32).at[:20].set(weights)
    dist = _dist_tc(stu_emb.T, cw48)
    parts = _sc_partials(
        b // _NW, dist, t1_prob.reshape(-1), t2_prob.reshape(-1), w32)
    return _final_sum(parts, 1.0 / b)[0, 0]
